# Initial kernel scaffold; baseline (speedup 1.0000x reference)
import jax
import jax.numpy as jnp
from jax.experimental import pallas as pl


def kernel(x, sigma, is_training, emb_table, W_pre, b_pre, W_pos, b_pos):
    raise NotImplementedError("write your pallas kernel here")


# speedup vs baseline: 1.5669x; 1.5669x over previous
import functools
import math

import jax
import jax.numpy as jnp
from jax import lax
from jax.experimental import pallas as pl
from jax.experimental.pallas import tpu as pltpu
from jax.experimental.pallas import tpu_sc as plsc

_NC = 2
_NS = 16
_NW = _NC * _NS

_VOCAB = 100000
_D = 1024
_NTOK = 4 * 4096
_B_PER_W = _NTOK // _NW
_CHUNK = 64
_NCHUNK = _B_PER_W // _CHUNK


def _gather_body(table_hbm, idx_hbm, out_hbm, idx_v, buf_v, sem):
    c = lax.axis_index("c")
    s = lax.axis_index("s")
    wid = s * _NC + c
    base = wid * _B_PER_W
    pltpu.sync_copy(idx_hbm.at[pl.ds(base, _B_PER_W)], idx_v)

    def body(g, carry):
        off = g * _CHUNK
        pltpu.async_copy(
            table_hbm.at[idx_v.at[pl.ds(off, _CHUNK)]], buf_v, sem
        ).wait()
        pltpu.sync_copy(buf_v, out_hbm.at[pl.ds(base + off, _CHUNK)])
        return carry

    lax.fori_loop(0, _NCHUNK, body, 0)


_gather = functools.partial(
    pl.kernel,
    mesh=plsc.VectorSubcoreMesh(core_axis_name="c", subcore_axis_name="s"),
    out_type=jax.ShapeDtypeStruct((_NTOK, _D), jnp.float32),
    scratch_types=[
        pltpu.VMEM((_B_PER_W,), jnp.int32),
        pltpu.VMEM((_CHUNK, _D), jnp.float32),
        pltpu.SemaphoreType.DMA,
    ],
)(_gather_body)


def _mlp_body(sigma_ref, wpre_ref, bpre_ref, wpos_ref, bpos_ref, out_ref):
    d_freq = wpre_ref.shape[0]
    half = d_freq // 2
    b = sigma_ref.shape[0]
    freqs = jnp.exp(
        (-math.log(10000.0) / half)
        * lax.broadcasted_iota(jnp.int32, (b, half), 1).astype(jnp.float32)
    )
    args = sigma_ref[:] * freqs
    t_freq = jnp.concatenate([jnp.cos(args), jnp.sin(args)], axis=-1)
    h = jnp.dot(t_freq, wpre_ref[:], preferred_element_type=jnp.float32)
    h = h + bpre_ref[:][None, :]
    h = h * jax.nn.sigmoid(h)
    h = jnp.dot(h, wpos_ref[:], preferred_element_type=jnp.float32)
    h = h + bpos_ref[:][None, :]
    out_ref[:] = h * jax.nn.sigmoid(h)


def _mlp(sigma, W_pre, b_pre, W_pos, b_pos):
    b = sigma.shape[0]
    d_model = W_pre.shape[1]
    return pl.pallas_call(
        _mlp_body,
        out_shape=jax.ShapeDtypeStruct((b, d_model), jnp.float32),
    )(sigma.reshape(b, 1), W_pre, b_pre, W_pos, b_pos)


def kernel(x, sigma, is_training, emb_table, W_pre, b_pre, W_pos, b_pos):
    idx = x.reshape(-1).astype(jnp.int32)
    y = _gather(emb_table, idx).reshape(x.shape + (emb_table.shape[1],))
    sigma_emb = _mlp(sigma, W_pre, b_pre, W_pos, b_pos)
    return (y, sigma_emb)
